# Initial kernel scaffold; baseline (speedup 1.0000x reference)
import jax
import jax.numpy as jnp
from jax.experimental import pallas as pl


def kernel(x_num, x_cat, params):
    raise NotImplementedError("write your pallas kernel here")


# speedup vs baseline: 5.8664x; 5.8664x over previous
import functools

import jax
import jax.numpy as jnp
from jax import lax
from jax.experimental import pallas as pl
from jax.experimental.pallas import tpu as pltpu
from jax.experimental.pallas import tpu_sc as plsc

_B = 1024
_DNUM = 32
_NCAT = 8
_CARD = 1000
_DT = 1024
_DH = 675
_NTOK = _DNUM + 1
_T = 41
_E = 8
_MID = 512
_EPS = 1e-5
_BC = 128
_SQRT_HALF = 0.7071067811865476
_NEG = -3.4e38


def _gelu(x):
    return 0.5 * x * (1.0 + lax.erf(x * _SQRT_HALF))


_GCH = 64


def _sc_gather(table, idx):
    info = plsc.get_sparse_core_info()
    nc, ns = info.num_cores, info.num_subcores
    nw = nc * ns
    n = idx.shape[0]
    bpw = n // nw
    mesh = plsc.VectorSubcoreMesh(core_axis_name="c", subcore_axis_name="s")

    @functools.partial(
        pl.kernel,
        mesh=mesh,
        out_type=jax.ShapeDtypeStruct((n, _DT), jnp.float32),
        scratch_types=[
            pltpu.VMEM((_GCH,), jnp.int32),
            pltpu.VMEM((_GCH, _DT), jnp.float32),
            pltpu.SemaphoreType.DMA,
        ],
    )
    def k(table_hbm, idx_hbm, out_hbm, idx_v, rows_v, sem):
        wid = lax.axis_index("s") * nc + lax.axis_index("c")
        base = wid * bpw
        for c in range(bpw // _GCH):
            off = base + c * _GCH
            pltpu.sync_copy(idx_hbm.at[pl.ds(off, _GCH)], idx_v)
            pltpu.async_copy(table_hbm.at[idx_v], rows_v, sem).wait()
            pltpu.sync_copy(rows_v, out_hbm.at[pl.ds(off, _GCH)])

    return k(table, idx)


def _s1_body(xn_ref, emb_ref, stats_ref, s0_ref, uv_ref, vv_ref, cvv_ref,
             catb_ref, lng_ref, lnb_ref, w0vt_ref, b0v_ref, sgug_ref,
             vconst_ref, w1ut_ref, cout_ref, hid_ref):
    xn = xn_ref[...]
    vw = stats_ref[0:1, :]
    cv_ = stats_ref[1:2, :]
    vc = stats_ref[2:3, :]
    rsig = lax.rsqrt(xn * xn * vw + 2.0 * xn * cv_ + vc + _EPS)
    alm = xn * rsig
    cvv = cvv_ref[...]
    bc = xn.shape[0]
    acc = jnp.zeros((bc, _DH), jnp.float32)
    for t in range(_NTOK):
        h = (alm[:, t:t + 1] * uv_ref[t:t + 1, :]
             + rsig[:, t:t + 1] * vv_ref[t:t + 1, :] + cvv)
        g = _gelu(h)
        mu = jnp.mean(g, axis=1, keepdims=True)
        s2 = jnp.mean(g * g, axis=1, keepdims=True)
        inv = lax.rsqrt(s2 - mu * mu + _EPS)
        acc = acc + s0_ref[0, t] * ((g - mu) * inv)
    lng = lng_ref[...]
    lnb = lnb_ref[...]
    xls = []
    for j in range(_NCAT):
        xc = emb_ref[:, j, :] + catb_ref[j:j + 1, :]
        m = jnp.mean(xc, axis=1, keepdims=True)
        v2 = jnp.mean(xc * xc, axis=1, keepdims=True) - m * m
        xls.append((xc - m) * lax.rsqrt(v2 + _EPS) * lng + lnb)
    xl2 = jnp.concatenate(xls, axis=0)
    h2 = jnp.dot(xl2, w0vt_ref[...],
                 preferred_element_type=jnp.float32) + b0v_ref[...]
    g2 = _gelu(h2)
    mu2 = jnp.mean(g2, axis=1, keepdims=True)
    s22 = jnp.mean(g2 * g2, axis=1, keepdims=True)
    inv2 = lax.rsqrt(s22 - mu2 * mu2 + _EPS)
    n2 = (g2 - mu2) * inv2
    for j in range(_NCAT):
        acc = acc + s0_ref[0, _NTOK + j] * n2[j * bc:(j + 1) * bc, :]
    v0 = acc * sgug_ref[...] + vconst_ref[...]
    hp = jnp.dot(v0, w1ut_ref[...],
                 preferred_element_type=jnp.float32) + cout_ref[...]
    m3 = jnp.mean(hp, axis=1, keepdims=True)
    v3 = jnp.mean(hp * hp, axis=1, keepdims=True) - m3 * m3
    hid_ref[...] = (hp - m3) * lax.rsqrt(v3 + _EPS) * lng + lnb


def _rep(shape):
    nd = len(shape)
    return pl.BlockSpec(shape, lambda i: (0,) * nd)


_S1_IN_SPECS = [
    pl.BlockSpec((_BC, _NTOK), lambda i: (i, 0)),
    pl.BlockSpec((_BC, _NCAT, _DT), lambda i: (i, 0, 0)),
    _rep((3, _NTOK)),
    pl.BlockSpec(memory_space=pltpu.SMEM),
    _rep((_NTOK, _DH)),
    _rep((_NTOK, _DH)),
    _rep((1, _DH)),
    _rep((_NCAT, _DT)),
    _rep((1, _DT)),
    _rep((1, _DT)),
    _rep((_DT, _DH)),
    _rep((1, _DH)),
    _rep((1, _DH)),
    _rep((1, _DH)),
    _rep((_DH, _DT)),
    _rep((1, _DT)),
]
_S1_OUT_SPEC = pl.BlockSpec((_BC, _DT), lambda i: (i, 0))
_S1_OUT_SHAPE = jax.ShapeDtypeStruct((_B, _DT), jnp.float32)


def _s2_body(hid_ref, wh1_ref, hb1_ref, ew1t_ref, eb1f_ref, ew2f_ref,
             eb2_ref, rw2t_ref, rb2_ref, gw2_ref, aw2_ref, sc2_ref, o_ref):
    hid = hid_ref[...]
    o1 = jnp.dot(hid, wh1_ref[...],
                 preferred_element_type=jnp.float32) + hb1_ref[...]
    gpre = o1[:, 0:_MID]
    rpre = o1[:, _MID:_MID + 256]
    apre = o1[:, _MID + 256:_MID + 512]
    base = o1[:, _MID + 512:_MID + 513]
    g2o = jnp.dot(jnp.maximum(gpre, 0.0), gw2_ref[...],
                  preferred_element_type=jnp.float32)
    rl = jnp.dot(jnp.maximum(rpre, 0.0), rw2t_ref[...],
                 preferred_element_type=jnp.float32) + rb2_ref[...]
    a = jnp.dot(jnp.maximum(apre, 0.0), aw2_ref[...],
                preferred_element_type=jnp.float32) + sc2_ref[0, 1]
    alpha = 1.0 / (1.0 + jnp.exp(-a))
    eh = jnp.maximum(jnp.dot(hid, ew1t_ref[...],
                             preferred_element_type=jnp.float32)
                     + eb1f_ref[...], 0.0)
    prod = eh * ew2f_ref[...]
    eos = [jnp.sum(prod[:, e * _MID:(e + 1) * _MID], axis=1, keepdims=True)
           for e in range(_E)]
    eo = jnp.concatenate(eos, axis=1) + eb2_ref[...]
    ids = lax.broadcasted_iota(jnp.int32, rl.shape, 1)
    m1 = jnp.max(rl, axis=1, keepdims=True)
    i1 = jnp.min(jnp.where(rl == m1, ids, _E), axis=1, keepdims=True)
    rl2 = jnp.where(ids == i1, _NEG, rl)
    m2 = jnp.max(rl2, axis=1, keepdims=True)
    i2 = jnp.min(jnp.where(rl2 == m2, ids, _E), axis=1, keepdims=True)
    e_ = jnp.exp(m2 - m1)
    w1 = 1.0 / (1.0 + e_)
    w2 = e_ * w1
    sel1 = jnp.sum(jnp.where(ids == i1, eo, 0.0), axis=1, keepdims=True)
    sel2 = jnp.sum(jnp.where(ids == i2, eo, 0.0), axis=1, keepdims=True)
    mix = w1 * sel1 + w2 * sel2
    o_ref[...] = base + g2o + sc2_ref[0, 0] + alpha * mix


_S2_IN_SPECS = [
    pl.BlockSpec((_BC, _DT), lambda i: (i, 0)),
    _rep((_DT, _MID + 513)),
    _rep((1, _MID + 513)),
    _rep((_DT, _E * _MID)),
    _rep((1, _E * _MID)),
    _rep((1, _E * _MID)),
    _rep((1, _E)),
    _rep((256, _E)),
    _rep((1, _E)),
    _rep((_MID, 1)),
    _rep((256, 1)),
    pl.BlockSpec(memory_space=pltpu.SMEM),
]
_S2_OUT_SPEC = pl.BlockSpec((_BC, 1), lambda i: (i, 0))
_S2_OUT_SHAPE = jax.ShapeDtypeStruct((_B, 1), jnp.float32)

_CPARAMS = pltpu.CompilerParams(dimension_semantics=("parallel",))

_s1_call = pl.pallas_call(
    _s1_body, grid=(_B // _BC,), in_specs=_S1_IN_SPECS,
    out_specs=_S1_OUT_SPEC, out_shape=_S1_OUT_SHAPE,
    compiler_params=_CPARAMS)

_s2_call = pl.pallas_call(
    _s2_body, grid=(_B // _BC,), in_specs=_S2_IN_SPECS,
    out_specs=_S2_OUT_SPEC, out_shape=_S2_OUT_SHAPE,
    compiler_params=_CPARAMS)


def _prep(p):
    w0v = p['W0'][_DH:, :]
    b0v = p['b0'][_DH:]
    w0u = p['W0'][:_DH, :]
    b0u = p['b0'][:_DH]
    w = p['tok_weight']
    c = jnp.concatenate(
        [jnp.zeros((1, _DT), jnp.float32), p['tok_bias'][:_DNUM]], axis=0)
    wt = w - w.mean(1, keepdims=True)
    ct = c - c.mean(1, keepdims=True)
    stats = jnp.stack([(wt * wt).mean(1), (wt * ct).mean(1),
                       (ct * ct).mean(1)])
    lng = p['ln_g']
    lnb = p['ln_b']
    uv = (wt * lng) @ w0v.T
    vv = (ct * lng) @ w0v.T
    cvv = (lnb @ w0v.T + b0v)[None]
    x0 = w[0]
    xln0 = (x0 - x0.mean()) / jnp.sqrt(x0.var() + _EPS) * lng + lnb
    u0 = _gelu(xln0 @ w0u.T + b0u)
    w1ut = (p['W1'] * u0[None, :]).T
    cout = (p['b1'] + w[0])[None]
    s0 = p['sgu_W'][0]
    vconst = (s0.sum() * p['sgu_ln_b'] + p['sgu_b'][0])[None]
    wh1 = jnp.concatenate(
        [p['gW1'].T, p['rW1'].T, p['aW1'].T, p['base_W'].T], axis=1)
    hb1 = jnp.concatenate(
        [p['gb1'], p['rb1'], p['ab1'], jnp.zeros((1,), jnp.float32)])[None]
    ew1t = p['eW1'].reshape(_E * _MID, _DT).T
    eb1f = p['eb1'].reshape(-1)[None]
    ew2f = p['eW2'].reshape(-1)[None]
    sc2 = jnp.stack([p['base_b'][0] + p['gb2'][0], p['ab2'][0]])[None]
    return dict(
        stats=stats, s0=s0[None], uv=uv, vv=vv, cvv=cvv,
        catb=p['tok_bias'][_DNUM:], lng=lng[None], lnb=lnb[None],
        w0vt=w0v.T, b0v=b0v[None], sgug=p['sgu_ln_g'][None],
        vconst=vconst, w1ut=w1ut, cout=cout,
        wh1=wh1, hb1=hb1, ew1t=ew1t, eb1f=eb1f, ew2f=ew2f,
        eb2=p['eb2'][None], rw2t=p['rW2'].T, rb2=p['rb2'][None],
        gw2=p['gW2'].T, aw2=p['aW2'].T, sc2=sc2)


def kernel(x_num, x_cat, params):
    b = x_num.shape[0]
    q = _prep(params)
    offs = (jnp.arange(_NCAT, dtype=jnp.int32) * _CARD)[None]
    idx = (x_cat.astype(jnp.int32) + offs).reshape(-1)
    emb = _sc_gather(params['cat_emb'], idx).reshape(b, _NCAT, _DT)
    xn = jnp.concatenate([jnp.ones((b, 1), jnp.float32), x_num], axis=1)
    hid = _s1_call(xn, emb, q['stats'], q['s0'], q['uv'], q['vv'], q['cvv'],
                   q['catb'], q['lng'], q['lnb'], q['w0vt'], q['b0v'],
                   q['sgug'], q['vconst'], q['w1ut'], q['cout'])
    out = _s2_call(hid, q['wh1'], q['hb1'], q['ew1t'], q['eb1f'], q['ew2f'],
                   q['eb2'], q['rw2t'], q['rb2'], q['gw2'], q['aw2'],
                   q['sc2'])
    return out[:, 0]
